# baseline (speedup 1.0000x reference)
import functools

import jax
import jax.numpy as jnp
from jax import lax
from jax.experimental import pallas as pl
from jax.experimental.pallas import tpu as pltpu
from jax.experimental.pallas import tpu_sc as plsc

B = 16384
N_CAT = 26
N_NUM = 13
VOCAB = 100000
D = 32
H = 128

NC = 2
NS = 16
NW = NC * NS
L = 16

ROWS = N_CAT * D
RPW = ROWS // NW
KG = ROWS // 8
CB = B // 128
CHUNK = 4096
NCHUNK = B // CHUNK
JROWS = CHUNK // 1024


def _gather_body(table_hbm, idx_hbm, out_hbm, row_v, idx_v, out_v, wsem):
    wid = lax.axis_index("s") * NC + lax.axis_index("c")

    def do_row(r, prev_field):
        k = wid * RPW + r
        i = k // D
        kg = k // 8
        dlk = lax.rem(k, 8)

        @pl.when(i != prev_field)
        def _():
            pltpu.sync_copy(idx_hbm.at[i, :], idx_v)

        rcopies = [
            pltpu.async_copy(
                table_hbm.at[k, pl.ds(p * 10000, 10000)],
                row_v.at[pl.ds(p * 10000, 10000)],
                wsem,
            )
            for p in range(10)
        ]
        for cp in rcopies:
            cp.wait()

        copies = []
        for c in range(NCHUNK):
            buf = c % 2
            if c >= 2:
                for cp in copies[(c - 2) * JROWS : (c - 1) * JROWS]:
                    cp.wait()
            for j in range(JROWS):
                def do_vec(t, _, _c=c, _j=j, _buf=buf):
                    for u in range(4):
                        tu = t * 4 + u
                        off = _c * CHUNK + _j * 1024 + tu * L
                        iv = idx_v[pl.ds(off, L)]
                        out_v[_buf, _j, tu // 8, pl.ds((tu % 8) * L, L)] = (
                            plsc.load_gather(row_v, [iv])
                        )
                    return 0

                lax.fori_loop(0, 1024 // (4 * L), do_vec, 0)
            for j in range(JROWS):
                copies.append(
                    pltpu.async_copy(
                        out_v.at[buf, j],
                        out_hbm.at[kg, pl.ds(c * (CHUNK // 128) + j * 8, 8), dlk, :],
                        wsem,
                    )
                )
        for cp in copies[(NCHUNK - 2) * JROWS :]:
            cp.wait()
        return i

    lax.fori_loop(0, RPW, do_row, jnp.int32(-1))


_gather_call = functools.partial(
    pl.kernel,
    out_type=jax.ShapeDtypeStruct((KG, CB, 8, 128), jnp.float32),
    mesh=plsc.VectorSubcoreMesh(core_axis_name="c", subcore_axis_name="s"),
    compiler_params=pltpu.CompilerParams(
        use_tc_tiling_on_sc=False, needs_layout_passes=False
    ),
    scratch_types=[
        pltpu.VMEM((VOCAB,), jnp.float32),
        pltpu.VMEM((B,), jnp.int32),
        pltpu.VMEM((2, JROWS, 8, 128), jnp.float32),
        pltpu.SemaphoreType.DMA,
    ],
)(_gather_body)


COLS_BLK = 1024
CBB = COLS_BLK // 128


def _mlp_body(c_ref, numt_ref, w1at_ref, w1bt_ref, b1_ref, w2t_ref, b2_ref, out_ref):
    hp = jax.lax.Precision.HIGHEST
    for cbl in range(CBB):
        m = c_ref[:, cbl, :, :].reshape(KG * 8, 128)
        h = jnp.dot(w1at_ref[...], m, preferred_element_type=jnp.float32, precision=hp)
        h += jnp.dot(
            w1bt_ref[...],
            numt_ref[:, pl.ds(cbl * 128, 128)],
            preferred_element_type=jnp.float32,
            precision=hp,
        )
        h = jnp.maximum(h + b1_ref[...], 0.0)
        y = jnp.dot(w2t_ref[...], h, preferred_element_type=jnp.float32, precision=hp)
        out_ref[:, pl.ds(cbl * 128, 128)] = jax.nn.sigmoid(y + b2_ref[...])


def _mlp(c4, numt, w1at, w1bt, b1c, w2t, b2c):
    grid = (B // COLS_BLK,)
    return pl.pallas_call(
        _mlp_body,
        grid=grid,
        in_specs=[
            pl.BlockSpec((KG, CBB, 8, 128), lambda j: (0, j, 0, 0)),
            pl.BlockSpec((N_NUM, COLS_BLK), lambda j: (0, j)),
            pl.BlockSpec((H, ROWS), lambda j: (0, 0)),
            pl.BlockSpec((H, N_NUM), lambda j: (0, 0)),
            pl.BlockSpec((H, 1), lambda j: (0, 0)),
            pl.BlockSpec((1, H), lambda j: (0, 0)),
            pl.BlockSpec((1, 1), lambda j: (0, 0)),
        ],
        out_specs=pl.BlockSpec((1, COLS_BLK), lambda j: (0, j)),
        out_shape=jax.ShapeDtypeStruct((1, B), jnp.float32),
    )(c4, numt, w1at, w1bt, b1c, w2t, b2c)


@jax.jit
def kernel(inputs, emb_tables, W1, b1, W2, b2):
    idx_t = inputs[:, :N_CAT].astype(jnp.int32).T
    num_t = inputs[:, N_CAT:].T
    table_lin = emb_tables.transpose(0, 2, 1).reshape(ROWS, VOCAB)
    c4 = _gather_call(table_lin, idx_t)
    w1at = W1[:ROWS].T
    w1bt = W1[ROWS:].T
    y = _mlp(c4, num_t, w1at, w1bt, b1.reshape(H, 1), W2.T, b2.reshape(1, 1))
    return y.reshape(B, 1)


# speedup vs baseline: 1.2587x; 1.2587x over previous
import functools

import jax
import jax.numpy as jnp
from jax import lax
from jax.experimental import pallas as pl
from jax.experimental.pallas import tpu as pltpu
from jax.experimental.pallas import tpu_sc as plsc

B = 16384
N_CAT = 26
N_NUM = 13
VOCAB = 100000
D = 32
H = 128

NC = 2
NS = 16
NW = NC * NS
L = 16

ROWS = N_CAT * D
RPW = ROWS // NW
KG = ROWS // 8
CB = B // 128
CHUNK = 4096
NCHUNK = B // CHUNK
JROWS = CHUNK // 1024


def _gather_body(table_hbm, idx_hbm, out_hbm, row_v, idx_v, out_v, wsem):
    wid = lax.axis_index("s") * NC + lax.axis_index("c")

    def do_row(r, prev_field):
        k = wid * RPW + r
        i = k // D
        kg = k // 8
        dlk = lax.rem(k, 8)

        @pl.when(i != prev_field)
        def _():
            pltpu.sync_copy(idx_hbm.at[i, :], idx_v)

        rcopies = [
            pltpu.async_copy(
                table_hbm.at[k, pl.ds(p * 10000, 10000)],
                row_v.at[pl.ds(p * 10000, 10000)],
                wsem,
            )
            for p in range(10)
        ]
        for cp in rcopies:
            cp.wait()

        copies = []
        for c in range(NCHUNK):
            buf = c % 2
            if c >= 2:
                for cp in copies[(c - 2) * JROWS : (c - 1) * JROWS]:
                    cp.wait()
            for j in range(JROWS):
                @plsc.parallel_loop(0, 1024 // L, unroll=8)
                def _(tu, _c=c, _j=j, _buf=buf):
                    off = _c * CHUNK + _j * 1024 + tu * L
                    iv = idx_v[pl.ds(off, L)]
                    out_v[_buf, _j, tu // 8, pl.ds((tu % 8) * L, L)] = (
                        plsc.load_gather(row_v, [iv])
                    )
            for j in range(JROWS):
                copies.append(
                    pltpu.async_copy(
                        out_v.at[buf, j],
                        out_hbm.at[kg, pl.ds(c * (CHUNK // 128) + j * 8, 8), dlk, :],
                        wsem,
                    )
                )
        for cp in copies[(NCHUNK - 2) * JROWS :]:
            cp.wait()
        return i

    lax.fori_loop(0, RPW, do_row, jnp.int32(-1))


_gather_call = functools.partial(
    pl.kernel,
    out_type=jax.ShapeDtypeStruct((KG, CB, 8, 128), jnp.float32),
    mesh=plsc.VectorSubcoreMesh(core_axis_name="c", subcore_axis_name="s"),
    compiler_params=pltpu.CompilerParams(
        use_tc_tiling_on_sc=False, needs_layout_passes=False
    ),
    scratch_types=[
        pltpu.VMEM((VOCAB,), jnp.float32),
        pltpu.VMEM((B,), jnp.int32),
        pltpu.VMEM((2, JROWS, 8, 128), jnp.float32),
        pltpu.SemaphoreType.DMA,
    ],
)(_gather_body)


COLS_BLK = 1024
CBB = COLS_BLK // 128


def _mlp_body(c_ref, numt_ref, w1at_ref, w1bt_ref, b1_ref, w2t_ref, b2_ref, out_ref):
    hp = jax.lax.Precision.HIGHEST
    for cbl in range(CBB):
        m = c_ref[:, cbl, :, :].reshape(KG * 8, 128)
        h = jnp.dot(w1at_ref[...], m, preferred_element_type=jnp.float32, precision=hp)
        h += jnp.dot(
            w1bt_ref[...],
            numt_ref[:, pl.ds(cbl * 128, 128)],
            preferred_element_type=jnp.float32,
            precision=hp,
        )
        h = jnp.maximum(h + b1_ref[...], 0.0)
        y = jnp.dot(w2t_ref[...], h, preferred_element_type=jnp.float32, precision=hp)
        out_ref[:, pl.ds(cbl * 128, 128)] = jax.nn.sigmoid(y + b2_ref[...])


def _mlp(c4, numt, w1at, w1bt, b1c, w2t, b2c):
    grid = (B // COLS_BLK,)
    return pl.pallas_call(
        _mlp_body,
        grid=grid,
        in_specs=[
            pl.BlockSpec((KG, CBB, 8, 128), lambda j: (0, j, 0, 0)),
            pl.BlockSpec((N_NUM, COLS_BLK), lambda j: (0, j)),
            pl.BlockSpec((H, ROWS), lambda j: (0, 0)),
            pl.BlockSpec((H, N_NUM), lambda j: (0, 0)),
            pl.BlockSpec((H, 1), lambda j: (0, 0)),
            pl.BlockSpec((1, H), lambda j: (0, 0)),
            pl.BlockSpec((1, 1), lambda j: (0, 0)),
        ],
        out_specs=pl.BlockSpec((1, COLS_BLK), lambda j: (0, j)),
        out_shape=jax.ShapeDtypeStruct((1, B), jnp.float32),
    )(c4, numt, w1at, w1bt, b1c, w2t, b2c)


@jax.jit
def kernel(inputs, emb_tables, W1, b1, W2, b2):
    idx_t = inputs[:, :N_CAT].astype(jnp.int32).T
    num_t = inputs[:, N_CAT:].T
    table_lin = emb_tables.transpose(0, 2, 1).reshape(ROWS, VOCAB)
    c4 = _gather_call(table_lin, idx_t)
    w1at = W1[:ROWS].T
    w1bt = W1[ROWS:].T
    y = _mlp(c4, num_t, w1at, w1bt, b1.reshape(H, 1), W2.T, b2.reshape(1, 1))
    return y.reshape(B, 1)


# speedup vs baseline: 1.2591x; 1.0003x over previous
import functools

import jax
import jax.numpy as jnp
from jax import lax
from jax.experimental import pallas as pl
from jax.experimental.pallas import tpu as pltpu
from jax.experimental.pallas import tpu_sc as plsc

B = 16384
N_CAT = 26
N_NUM = 13
VOCAB = 100000
D = 32
H = 128

NC = 2
NS = 16
NW = NC * NS
L = 16

ROWS = N_CAT * D
RPW = ROWS // NW
KG = ROWS // 8
CB = B // 128
CHUNK = 4096
NCHUNK = B // CHUNK
JROWS = CHUNK // 1024


def _gather_body(table_hbm, idx_hbm, out_hbm, row_v, idx_v, out_v, wsem):
    wid = lax.axis_index("s") * NC + lax.axis_index("c")

    def do_row(r, prev_field):
        k = wid * RPW + r
        i = k // D
        kg = k // 8
        dlk = lax.rem(k, 8)

        @pl.when(i != prev_field)
        def _():
            pltpu.sync_copy(idx_hbm.at[i, :], idx_v)

        rcopies = [
            pltpu.async_copy(
                table_hbm.at[k, pl.ds(p * 10000, 10000)],
                row_v.at[pl.ds(p * 10000, 10000)],
                wsem,
            )
            for p in range(10)
        ]
        for cp in rcopies:
            cp.wait()

        copies = []
        for c in range(NCHUNK):
            buf = c % 2
            if c >= 2:
                for cp in copies[(c - 2) * JROWS : (c - 1) * JROWS]:
                    cp.wait()
            for j in range(JROWS):
                @plsc.parallel_loop(0, 1024 // L, unroll=16)
                def _(tu, _c=c, _j=j, _buf=buf):
                    off = _c * CHUNK + _j * 1024 + tu * L
                    iv = idx_v[pl.ds(off, L)]
                    out_v[_buf, _j, tu // 8, pl.ds((tu % 8) * L, L)] = (
                        plsc.load_gather(row_v, [iv])
                    )
            for j in range(JROWS):
                copies.append(
                    pltpu.async_copy(
                        out_v.at[buf, j],
                        out_hbm.at[kg, pl.ds(c * (CHUNK // 128) + j * 8, 8), dlk, :],
                        wsem,
                    )
                )
        for cp in copies[(NCHUNK - 2) * JROWS :]:
            cp.wait()
        return i

    lax.fori_loop(0, RPW, do_row, jnp.int32(-1))


_gather_call = functools.partial(
    pl.kernel,
    out_type=jax.ShapeDtypeStruct((KG, CB, 8, 128), jnp.float32),
    mesh=plsc.VectorSubcoreMesh(core_axis_name="c", subcore_axis_name="s"),
    compiler_params=pltpu.CompilerParams(
        use_tc_tiling_on_sc=False, needs_layout_passes=False
    ),
    scratch_types=[
        pltpu.VMEM((VOCAB,), jnp.float32),
        pltpu.VMEM((B,), jnp.int32),
        pltpu.VMEM((2, JROWS, 8, 128), jnp.float32),
        pltpu.SemaphoreType.DMA,
    ],
)(_gather_body)


COLS_BLK = 1024
CBB = COLS_BLK // 128


def _mlp_body(c_ref, numt_ref, w1at_ref, w1bt_ref, b1_ref, w2t_ref, b2_ref, out_ref):
    hp = jax.lax.Precision.HIGHEST
    for cbl in range(CBB):
        m = c_ref[:, cbl, :, :].reshape(KG * 8, 128)
        h = jnp.dot(w1at_ref[...], m, preferred_element_type=jnp.float32, precision=hp)
        h += jnp.dot(
            w1bt_ref[...],
            numt_ref[:, pl.ds(cbl * 128, 128)],
            preferred_element_type=jnp.float32,
            precision=hp,
        )
        h = jnp.maximum(h + b1_ref[...], 0.0)
        y = jnp.dot(w2t_ref[...], h, preferred_element_type=jnp.float32, precision=hp)
        out_ref[:, pl.ds(cbl * 128, 128)] = jax.nn.sigmoid(y + b2_ref[...])


def _mlp(c4, numt, w1at, w1bt, b1c, w2t, b2c):
    grid = (B // COLS_BLK,)
    return pl.pallas_call(
        _mlp_body,
        grid=grid,
        in_specs=[
            pl.BlockSpec((KG, CBB, 8, 128), lambda j: (0, j, 0, 0)),
            pl.BlockSpec((N_NUM, COLS_BLK), lambda j: (0, j)),
            pl.BlockSpec((H, ROWS), lambda j: (0, 0)),
            pl.BlockSpec((H, N_NUM), lambda j: (0, 0)),
            pl.BlockSpec((H, 1), lambda j: (0, 0)),
            pl.BlockSpec((1, H), lambda j: (0, 0)),
            pl.BlockSpec((1, 1), lambda j: (0, 0)),
        ],
        out_specs=pl.BlockSpec((1, COLS_BLK), lambda j: (0, j)),
        out_shape=jax.ShapeDtypeStruct((1, B), jnp.float32),
    )(c4, numt, w1at, w1bt, b1c, w2t, b2c)


@jax.jit
def kernel(inputs, emb_tables, W1, b1, W2, b2):
    idx_t = inputs[:, :N_CAT].astype(jnp.int32).T
    num_t = inputs[:, N_CAT:].T
    table_lin = emb_tables.transpose(0, 2, 1).reshape(ROWS, VOCAB)
    c4 = _gather_call(table_lin, idx_t)
    w1at = W1[:ROWS].T
    w1bt = W1[ROWS:].T
    y = _mlp(c4, num_t, w1at, w1bt, b1.reshape(H, 1), W2.T, b2.reshape(1, 1))
    return y.reshape(B, 1)
